# baseline (speedup 1.0000x reference)
import functools

import jax
import jax.numpy as jnp
from jax import lax
from jax.experimental import pallas as pl
from jax.experimental.pallas import tpu as pltpu
from jax.experimental.pallas import tpu_sc as plsc

H = 16
T = 961
N = 256
NW = 32
ROWS = N // NW
GROUPS = ROWS * N // 16
HH = H // 2

_mesh = plsc.VectorSubcoreMesh(core_axis_name="c", subcore_axis_name="s")


@functools.partial(
    pl.kernel,
    mesh=_mesh,
    out_type=jax.ShapeDtypeStruct((H, N, N), jnp.float32),
    scratch_types=[
        pltpu.VMEM((H, T), jnp.float32),
        pltpu.VMEM((ROWS, N), jnp.int32),
        pltpu.VMEM((H, ROWS, N), jnp.float32),
        pltpu.SemaphoreType.DMA,
        pltpu.SemaphoreType.DMA,
        pltpu.SemaphoreType.DMA,
        pltpu.SemaphoreType.DMA,
    ],
    compiler_params=pltpu.CompilerParams(
        needs_layout_passes=False,
        disable_bounds_checks=True,
    ),
)
def _bias_kernel(tab_hbm, idx_hbm, out_hbm, tab_v, idx_v, out_v,
                 sem_t1, sem_t2, sem_i, sem_o):
    wid = lax.axis_index("s") * 2 + lax.axis_index("c")
    row0 = wid * ROWS
    cp_t1 = pltpu.async_copy(
        tab_hbm.at[pl.ds(0, HH), :], tab_v.at[pl.ds(0, HH), :], sem_t1)
    cp_t2 = pltpu.async_copy(
        tab_hbm.at[pl.ds(HH, HH), :], tab_v.at[pl.ds(HH, HH), :], sem_t2)
    cp_i = pltpu.async_copy(idx_hbm.at[pl.ds(row0, ROWS), :], idx_v, sem_i)
    cp_t1.wait()
    cp_i.wait()

    @plsc.parallel_loop(0, GROUPS, unroll=1)
    def body0(g):
        r = g >> 4
        c = (g & 15) * 16
        iv = idx_v[r, pl.ds(c, 16)]
        for h in range(HH):
            hv = jnp.full((16,), h, dtype=jnp.int32)
            out_v[h, r, pl.ds(c, 16)] = plsc.load_gather(tab_v, [hv, iv])

    cp_o = pltpu.async_copy(
        out_v.at[pl.ds(0, HH), :, :],
        out_hbm.at[pl.ds(0, HH), pl.ds(row0, ROWS), :],
        sem_o,
    )
    cp_t2.wait()

    @plsc.parallel_loop(0, GROUPS, unroll=1)
    def body1(g):
        r = g >> 4
        c = (g & 15) * 16
        iv = idx_v[r, pl.ds(c, 16)]
        for h in range(HH, H):
            hv = jnp.full((16,), h, dtype=jnp.int32)
            out_v[h, r, pl.ds(c, 16)] = plsc.load_gather(tab_v, [hv, iv])

    cp_o.wait()
    pltpu.sync_copy(
        out_v.at[pl.ds(HH, HH), :, :],
        out_hbm.at[pl.ds(HH, HH), pl.ds(row0, ROWS), :],
    )


def kernel(table, index):
    tab_t = jnp.transpose(table)
    out = _bias_kernel(tab_t, index.astype(jnp.int32))
    return out.reshape(1, H, N, N)


# speedup vs baseline: 1.1027x; 1.1027x over previous
import functools

import jax
import jax.numpy as jnp
from jax import lax
from jax.experimental import pallas as pl
from jax.experimental.pallas import tpu as pltpu
from jax.experimental.pallas import tpu_sc as plsc

H = 16
T = 961
N = 256
NW = 32
ROWS = N // NW
GROUPS = ROWS * N // 16

_mesh = plsc.VectorSubcoreMesh(core_axis_name="c", subcore_axis_name="s")


@functools.partial(
    pl.kernel,
    mesh=_mesh,
    out_type=jax.ShapeDtypeStruct((H, N, N), jnp.float32),
    scratch_types=[
        pltpu.VMEM((H, T), jnp.float32),
        pltpu.VMEM((ROWS, N), jnp.int32),
        pltpu.VMEM((H, ROWS, N), jnp.float32),
        pltpu.SemaphoreType.DMA,
        pltpu.SemaphoreType.DMA,
    ],
    compiler_params=pltpu.CompilerParams(
        needs_layout_passes=False,
        disable_bounds_checks=True,
        disable_semaphore_checks=True,
    ),
)
def _bias_kernel(tab_hbm, idx_hbm, out_hbm, tab_v, idx_v, out_v, sem_t, sem_i):
    wid = lax.axis_index("s") * 2 + lax.axis_index("c")
    row0 = wid * ROWS
    cp_t = pltpu.async_copy(tab_hbm, tab_v, sem_t)
    cp_i = pltpu.async_copy(idx_hbm.at[pl.ds(row0, ROWS), :], idx_v, sem_i)
    cp_t.wait()
    cp_i.wait()

    @plsc.parallel_loop(0, GROUPS, unroll=1)
    def body(g):
        r = g >> 4
        c = (g & 15) * 16
        iv = idx_v[r, pl.ds(c, 16)]
        for h in range(H):
            hv = jnp.full((16,), h, dtype=jnp.int32)
            out_v[h, r, pl.ds(c, 16)] = plsc.load_gather(tab_v, [hv, iv])

    pltpu.sync_copy(out_v, out_hbm.at[:, pl.ds(row0, ROWS), :])


def kernel(table, index):
    tab_t = jnp.transpose(table)
    out = _bias_kernel(tab_t, index.astype(jnp.int32))
    return out.reshape(1, H, N, N)


# speedup vs baseline: 1.1573x; 1.0495x over previous
import functools

import jax
import jax.numpy as jnp
from jax import lax
from jax.experimental import pallas as pl
from jax.experimental.pallas import tpu as pltpu
from jax.experimental.pallas import tpu_sc as plsc

H = 16
T = 961
N = 256
HH = 8
ROWS = 16
GROUPS = ROWS * N // 16

_mesh = plsc.VectorSubcoreMesh(core_axis_name="c", subcore_axis_name="s")


@functools.partial(
    pl.kernel,
    mesh=_mesh,
    out_type=jax.ShapeDtypeStruct((H, N, N), jnp.float32),
    scratch_types=[
        pltpu.VMEM((HH, T), jnp.float32),
        pltpu.VMEM((ROWS, N), jnp.int32),
        pltpu.VMEM((HH, ROWS, N), jnp.float32),
        pltpu.SemaphoreType.DMA,
        pltpu.SemaphoreType.DMA,
    ],
    compiler_params=pltpu.CompilerParams(
        needs_layout_passes=False,
        disable_bounds_checks=True,
        disable_semaphore_checks=True,
    ),
)
def _bias_kernel(tab_hbm, idx_hbm, out_hbm, tab_v, idx_v, out_v, sem_t, sem_i):
    wid = lax.axis_index("s") * 2 + lax.axis_index("c")
    h0 = (wid & 1) * HH
    row0 = (wid >> 1) * ROWS
    cp_t = pltpu.async_copy(tab_hbm.at[pl.ds(h0, HH), :], tab_v, sem_t)
    cp_i = pltpu.async_copy(idx_hbm.at[pl.ds(row0, ROWS), :], idx_v, sem_i)
    cp_t.wait()
    cp_i.wait()

    @plsc.parallel_loop(0, GROUPS, unroll=1)
    def body(g):
        r = g >> 4
        c = (g & 15) * 16
        iv = idx_v[r, pl.ds(c, 16)]
        for h in range(HH):
            hv = jnp.full((16,), h, dtype=jnp.int32)
            out_v[h, r, pl.ds(c, 16)] = plsc.load_gather(tab_v, [hv, iv])

    pltpu.sync_copy(out_v, out_hbm.at[pl.ds(h0, HH), pl.ds(row0, ROWS), :])


def kernel(table, index):
    tab_t = jnp.transpose(table)
    out = _bias_kernel(tab_t, index.astype(jnp.int32))
    return out.reshape(1, H, N, N)
